# baseline (speedup 1.0000x reference)
import functools

import jax
import jax.numpy as jnp
from jax import lax
from jax.experimental import pallas as pl
from jax.experimental.pallas import tpu as pltpu
from jax.experimental.pallas import tpu_sc as plsc

N, H, W = 16, 224, 224
C = 96
P = H * W
L = 16
NC, NS = 2, 16
NW = NC * NS
WPIX = (N * P) // NW
TP = 256
TASKS = WPIX // TP
CHUNKS = TP // L


def kernel(x):
    mesh = plsc.VectorSubcoreMesh(core_axis_name="c", subcore_axis_name="s")

    @functools.partial(
        pl.kernel,
        mesh=mesh,
        compiler_params=pltpu.CompilerParams(
            use_tc_tiling_on_sc=False, needs_layout_passes=False
        ),
        out_type=jax.ShapeDtypeStruct((N, C, P), jnp.float32),
        scratch_types=[
            pltpu.VMEM((WPIX,), jnp.int32),
            pltpu.VMEM((C, TP), jnp.float32),
            pltpu.VMEM((C, TP), jnp.float32),
            pltpu.SemaphoreType.DMA,
            pltpu.SemaphoreType.DMA,
        ],
    )
    def k(x_hbm, out_hbm, x_v, oh0, oh1, sem0, sem1):
        wid = lax.axis_index("s") * NC + lax.axis_index("c")
        n = wid // 2
        p_base = (wid % 2) * WPIX

        zeros = jnp.zeros((L,), jnp.float32)
        ones = jnp.ones((L,), jnp.float32)
        lane = lax.broadcasted_iota(jnp.int32, (L,), 0)
        bufs = (oh0, oh1)
        sems = (sem0, sem1)

        pltpu.sync_copy(x_hbm.at[pl.ds(wid * WPIX, WPIX)], x_v)

        def zbody(c, carry):
            for buf in bufs:
                for j in range(CHUNKS):
                    buf[c, pl.ds(j * L, L)] = zeros
            return carry

        lax.fori_loop(0, C, zbody, 0)

        def scatter_task(buf, i, val_vec):
            for j in range(CHUNKS):
                vals = x_v[pl.ds(i * TP + j * L, L)]
                plsc.store_scatter(buf, [vals, lane + j * L], val_vec)

        def gbody(g, carry):
            for b in range(2):
                i = g * 2 + b
                buf, sem = bufs[b], sems[b]

                @pl.when(g > 0)
                def _():
                    pltpu.make_async_copy(
                        buf, out_hbm.at[n, :, pl.ds(p_base, TP)], sem
                    ).wait()
                    scatter_task(buf, i - 2, zeros)

                scatter_task(buf, i, ones)
                pltpu.async_copy(
                    buf, out_hbm.at[n, :, pl.ds(p_base + i * TP, TP)], sem
                )
            return carry

        lax.fori_loop(0, TASKS // 2, gbody, 0)

        for b in range(2):
            pltpu.make_async_copy(
                bufs[b], out_hbm.at[n, :, pl.ds(p_base, TP)], sems[b]
            ).wait()

    return k(x.reshape(N * P)).reshape(N, C, H, W)
